# Initial kernel scaffold; baseline (speedup 1.0000x reference)
import jax
import jax.numpy as jnp
from jax.experimental import pallas as pl


def kernel(x, ln1_g, ln1_b, Wq, bq, Wkv, bkv, Wp, bp, ln2_g, ln2_b, Wf1, bf1, w_skip, w_dw, b_dw, Wf2, bf2, idx_token, H, W, H_init, W_init):
    raise NotImplementedError("write your pallas kernel here")


# speedup vs baseline: 5.1005x; 5.1005x over previous
import functools
import math

import jax
import jax.numpy as jnp
from jax import lax
from jax.experimental import pallas as pl
from jax.experimental.pallas import tpu as pltpu
from jax.experimental.pallas import tpu_sc as plsc

B, N, C = 4, 4096, 96
HEADS, HD = 2, 48
HID = 384
G = 128
NP = G * G
M = B * NP
SCALE = HD ** -0.5
C1 = 1.0 / (1.0 + 1e-6)

_NC, _NS = 2, 16
NW = _NC * _NS
_K = 64
CP = 128


def _sc_gather(table, idxg, D):
    m_per_w = M // NW
    nchunk = m_per_w // _K

    def body(tab_hbm, idx_hbm, out_hbm, idx_v, rows_v, sem):
        wid = lax.axis_index("s") * _NC + lax.axis_index("c")
        base = wid * m_per_w

        def chunk(i, carry):
            off = base + i * _K
            pltpu.sync_copy(idx_hbm.at[pl.ds(off, _K)], idx_v)
            pltpu.async_copy(tab_hbm.at[idx_v], rows_v, sem).wait()
            pltpu.sync_copy(rows_v, out_hbm.at[pl.ds(off, _K)])
            return carry

        lax.fori_loop(0, nchunk, chunk, 0)

    mesh = plsc.VectorSubcoreMesh(core_axis_name="c", subcore_axis_name="s")
    return pl.kernel(
        body,
        out_type=jax.ShapeDtypeStruct((M, D), jnp.float32),
        mesh=mesh,
        scratch_types=[
            pltpu.VMEM((_K,), jnp.int32),
            pltpu.VMEM((_K, D), jnp.float32),
            pltpu.SemaphoreType.DMA,
        ],
    )(table, idxg)


def _sc_scatter(s0, s1, s2, idxl, zeros_t):
    per_sub = NP // _NS
    nchunk = per_sub // _K
    rows_z = N // _NS

    def body(s0_hbm, s1_hbm, s2_hbm, idx_hbm, zeros_hbm,
             o0_hbm, o1_hbm, o2_hbm,
             idx_v, r0_v, r1_v, r2_v, a0, a1, a2):
        c = lax.axis_index("c")
        s = lax.axis_index("s")
        accs = (a0, a1, a2)
        outs = (o0_hbm, o1_hbm, o2_hbm)
        slabs = (s0_hbm, s1_hbm, s2_hbm)
        rows = (r0_v, r1_v, r2_v)
        for bi in range(B // _NC):
            b = c + _NC * bi
            for acc in accs:
                pltpu.sync_copy(zeros_hbm, acc.at[pl.ds(s * rows_z, rows_z)])
            plsc.subcore_barrier()

            def chunk(i, carry):
                off = b * NP + s * per_sub + i * _K
                pltpu.sync_copy(idx_hbm.at[pl.ds(off, _K)], idx_v)
                for slab, rv, acc in zip(slabs, rows, accs):
                    pltpu.sync_copy(slab.at[pl.ds(off, _K)], rv)
                    pltpu.sync_copy(rv, acc.at[idx_v], add=True)
                return carry

            lax.fori_loop(0, nchunk, chunk, 0)
            plsc.subcore_barrier()
            for acc, out in zip(accs, outs):
                pltpu.sync_copy(acc.at[pl.ds(s * rows_z, rows_z)],
                                out.at[pl.ds(b * N + s * rows_z, rows_z)])
            plsc.subcore_barrier()

    mesh = plsc.VectorSubcoreMesh(core_axis_name="c", subcore_axis_name="s")
    sds = jax.ShapeDtypeStruct((B * N, CP), jnp.float32)
    return pl.kernel(
        body,
        out_type=(sds, sds, sds),
        mesh=mesh,
        scratch_types=[
            pltpu.VMEM((_K,), jnp.int32),
            pltpu.VMEM((_K, CP), jnp.float32),
            pltpu.VMEM((_K, CP), jnp.float32),
            pltpu.VMEM((_K, CP), jnp.float32),
            pltpu.VMEM_SHARED((N, CP), jnp.float32),
            pltpu.VMEM_SHARED((N, CP), jnp.float32),
            pltpu.VMEM_SHARED((N, CP), jnp.float32),
        ],
    )(s0, s1, s2, idxl, zeros_t)


def _sc_count(idxl, ones_t, zeros_t):
    per_sub = NP // _NS
    nchunk = per_sub // _K
    rows_z = N // _NS

    def body(idx_hbm, ones_hbm, zeros_hbm, cnt_hbm, idx_v, ones_v, cnt_sh):
        c = lax.axis_index("c")
        s = lax.axis_index("s")
        pltpu.sync_copy(ones_hbm, ones_v)
        for bi in range(B // _NC):
            b = c + _NC * bi
            pltpu.sync_copy(zeros_hbm, cnt_sh.at[pl.ds(s * rows_z, rows_z)])
            plsc.subcore_barrier()

            def chunk(i, carry):
                off = b * NP + s * per_sub + i * _K
                pltpu.sync_copy(idx_hbm.at[pl.ds(off, _K)], idx_v)
                pltpu.sync_copy(ones_v, cnt_sh.at[idx_v], add=True)
                return carry

            lax.fori_loop(0, nchunk, chunk, 0)
            plsc.subcore_barrier()
            pltpu.sync_copy(cnt_sh.at[pl.ds(s * rows_z, rows_z)],
                            cnt_hbm.at[pl.ds(b * N + s * rows_z, rows_z)])
            plsc.subcore_barrier()

    mesh = plsc.VectorSubcoreMesh(core_axis_name="c", subcore_axis_name="s")
    return pl.kernel(
        body,
        out_type=jax.ShapeDtypeStruct((B * N, CP), jnp.float32),
        mesh=mesh,
        scratch_types=[
            pltpu.VMEM((_K,), jnp.int32),
            pltpu.VMEM((_K, CP), jnp.float32),
            pltpu.VMEM_SHARED((N, CP), jnp.float32),
        ],
    )(idxl, ones_t, zeros_t)


def _ln(x_ref, g_ref, b_ref):
    xv = x_ref[...]
    m = jnp.mean(xv, axis=1, keepdims=True)
    d = xv - m
    v = jnp.mean(d * d, axis=1, keepdims=True)
    return d * lax.rsqrt(v + 1e-5) * g_ref[...] + b_ref[...]


def _t1_body(x_ref, g_ref, b_ref, w_ref, bias_ref, xn_ref, q_ref):
    xn = _ln(x_ref, g_ref, b_ref)
    xn_ref[...] = jnp.concatenate(
        [xn, jnp.zeros((xn.shape[0], CP - C), jnp.float32)], axis=1)
    q_ref[...] = (jnp.dot(xn, w_ref[...], preferred_element_type=jnp.float32)
                  + bias_ref[...])


def _t1(xf, g, bvec, Wq, bq):
    R = 512
    grid = (B * N) // R
    return pl.pallas_call(
        _t1_body,
        grid=(grid,),
        in_specs=[
            pl.BlockSpec((R, C), lambda i: (i, 0)),
            pl.BlockSpec((1, C), lambda i: (0, 0)),
            pl.BlockSpec((1, C), lambda i: (0, 0)),
            pl.BlockSpec((C, C), lambda i: (0, 0)),
            pl.BlockSpec((1, C), lambda i: (0, 0)),
        ],
        out_specs=[
            pl.BlockSpec((R, CP), lambda i: (i, 0)),
            pl.BlockSpec((R, C), lambda i: (i, 0)),
        ],
        out_shape=[
            jax.ShapeDtypeStruct((B * N, CP), jnp.float32),
            jax.ShapeDtypeStruct((B * N, C), jnp.float32),
        ],
    )(xf, g, bvec, Wq, bq)


def _t3_body(x_ref, g_ref, b_ref, w_ref, bias_ref, y_ref):
    xn = _ln(x_ref, g_ref, b_ref)
    y_ref[...] = (jnp.dot(xn, w_ref[...], preferred_element_type=jnp.float32)
                  + bias_ref[...])


def _t3(x2f, g, bvec, Wf1, bf1):
    R = 512
    grid = (B * N) // R
    return pl.pallas_call(
        _t3_body,
        grid=(grid,),
        in_specs=[
            pl.BlockSpec((R, C), lambda i: (i, 0)),
            pl.BlockSpec((1, C), lambda i: (0, 0)),
            pl.BlockSpec((1, C), lambda i: (0, 0)),
            pl.BlockSpec((C, HID), lambda i: (0, 0)),
            pl.BlockSpec((1, HID), lambda i: (0, 0)),
        ],
        out_specs=pl.BlockSpec((R, HID), lambda i: (i, 0)),
        out_shape=jax.ShapeDtypeStruct((B * N, HID), jnp.float32),
    )(x2f, g, bvec, Wf1, bf1)


def _t2_body(x_ref, q_ref, gkv_ref, wkv_ref, bkv_ref, wp_ref, bp_ref, x2_ref):
    def add(j, acc):
        return acc + gkv_ref[pl.ds(j * 256, 256), :C]

    ksum = lax.fori_loop(0, 64, add, jnp.zeros((256, C), jnp.float32))
    kv_tok = ksum * (C1 / 64.0)
    kv = (jnp.dot(kv_tok, wkv_ref[...], preferred_element_type=jnp.float32)
          + bkv_ref[...])
    q = q_ref[...]
    outs = []
    for h in range(HEADS):
        k_h = kv[:, h * HD:(h + 1) * HD]
        v_h = kv[:, C + h * HD:C + (h + 1) * HD]
        q_h = q[:, h * HD:(h + 1) * HD]
        logits = lax.dot_general(q_h, k_h, (((1,), (1,)), ((), ())),
                                 preferred_element_type=jnp.float32) * SCALE
        mx = jnp.max(logits, axis=1, keepdims=True)
        e = jnp.exp(logits - mx)
        p = e / jnp.sum(e, axis=1, keepdims=True)
        outs.append(jnp.dot(p, v_h, preferred_element_type=jnp.float32))
    o = jnp.concatenate(outs, axis=1)
    x2_ref[...] = (x_ref[...]
                   + jnp.dot(o, wp_ref[...], preferred_element_type=jnp.float32)
                   + bp_ref[...])


def _t2(xf, qf, gkv, Wkv, bkv, Wp, bp):
    return pl.pallas_call(
        _t2_body,
        grid=(B,),
        in_specs=[
            pl.BlockSpec((N, C), lambda b: (b, 0)),
            pl.BlockSpec((N, C), lambda b: (b, 0)),
            pl.BlockSpec((NP, CP), lambda b: (b, 0)),
            pl.BlockSpec((C, 2 * C), lambda b: (0, 0)),
            pl.BlockSpec((1, 2 * C), lambda b: (0, 0)),
            pl.BlockSpec((C, C), lambda b: (0, 0)),
            pl.BlockSpec((1, C), lambda b: (0, 0)),
        ],
        out_specs=pl.BlockSpec((N, C), lambda b: (b, 0)),
        out_shape=jax.ShapeDtypeStruct((B * N, C), jnp.float32),
    )(xf, qf, gkv, Wkv, bkv, Wp, bp)


def _t4_body(up_ref, mid_ref, dn_ref, w_ref, bd_ref, o0_ref, o1_ref, o2_ref):
    t = pl.program_id(1)
    nt = pl.num_programs(1)
    zrow = jnp.zeros((1, G, HID), jnp.float32)
    prev = jnp.where(t > 0, up_ref[0, 15:16], zrow)
    nxt = jnp.where(t < nt - 1, dn_ref[0, 0:1], zrow)
    padded = jnp.concatenate([prev, mid_ref[0], nxt], axis=0)
    acc = jnp.zeros((16, G, HID), jnp.float32)
    zcol = jnp.zeros((16, 1, HID), jnp.float32)
    for dr in range(3):
        rows = padded[dr:dr + 16]
        for dc in range(3):
            if dc == 0:
                sh = jnp.concatenate([zcol, rows[:, :-1]], axis=1)
            elif dc == 1:
                sh = rows
            else:
                sh = jnp.concatenate([rows[:, 1:], zcol], axis=1)
            acc = acc + sh * w_ref[dr * 3 + dc]
    acc = acc + bd_ref[0]
    o0_ref[0] = acc[:, :, 0:CP]
    o1_ref[0] = acc[:, :, CP:2 * CP]
    o2_ref[0] = acc[:, :, 2 * CP:3 * CP]


def _t4(ggrid, wdw9, bdw):
    RT = 16
    nt = G // RT
    spec = lambda f: pl.BlockSpec((1, RT, G, HID), f)
    ospec = pl.BlockSpec((1, RT, G, CP), lambda b, t: (b, t, 0, 0))
    osds = jax.ShapeDtypeStruct((B, G, G, CP), jnp.float32)
    return pl.pallas_call(
        _t4_body,
        grid=(B, nt),
        in_specs=[
            spec(lambda b, t: (b, jnp.maximum(t - 1, 0), 0, 0)),
            spec(lambda b, t: (b, t, 0, 0)),
            spec(lambda b, t: (b, jnp.minimum(t + 1, nt - 1), 0, 0)),
            pl.BlockSpec((9, HID), lambda b, t: (0, 0)),
            pl.BlockSpec((1, HID), lambda b, t: (0, 0)),
        ],
        out_specs=[ospec, ospec, ospec],
        out_shape=[osds, osds, osds],
    )(ggrid, ggrid, ggrid, wdw9, bdw)


def _t5_body(hdn_ref, h0_ref, h1_ref, h2_ref, cnt_ref, x2_ref, wskip_ref,
             wf2_ref, bf2_ref, out_ref):
    cntv = cnt_ref[:, 0:1]
    hsum = jnp.concatenate([h0_ref[...], h1_ref[...], h2_ref[...]], axis=1)
    htok = hsum / (cntv + 1e-6)
    a = hdn_ref[...] * wskip_ref[...] + htok
    gl = a * 0.5 * (1.0 + lax.erf(a * (2.0 ** -0.5)))
    out_ref[...] = (x2_ref[...]
                    + jnp.dot(gl, wf2_ref[...], preferred_element_type=jnp.float32)
                    + bf2_ref[...])


def _t5(hdn, h0, h1, h2, cnt, x2f, wskip, Wf2, bf2):
    R = 512
    grid = (B * N) // R
    return pl.pallas_call(
        _t5_body,
        grid=(grid,),
        in_specs=[
            pl.BlockSpec((R, HID), lambda i: (i, 0)),
            pl.BlockSpec((R, CP), lambda i: (i, 0)),
            pl.BlockSpec((R, CP), lambda i: (i, 0)),
            pl.BlockSpec((R, CP), lambda i: (i, 0)),
            pl.BlockSpec((R, CP), lambda i: (i, 0)),
            pl.BlockSpec((R, C), lambda i: (i, 0)),
            pl.BlockSpec((1, HID), lambda i: (0, 0)),
            pl.BlockSpec((HID, C), lambda i: (0, 0)),
            pl.BlockSpec((1, C), lambda i: (0, 0)),
        ],
        out_specs=pl.BlockSpec((R, C), lambda i: (i, 0)),
        out_shape=jax.ShapeDtypeStruct((B * N, C), jnp.float32),
    )(hdn, h0, h1, h2, cnt, x2f, wskip, Wf2, bf2)


def kernel(x, ln1_g, ln1_b, Wq, bq, Wkv, bkv, Wp, bp, ln2_g, ln2_b, Wf1, bf1,
           w_skip, w_dw, b_dw, Wf2, bf2, idx_token, H, W, H_init, W_init):
    xf = x.reshape(B * N, C)
    idx = idx_token.astype(jnp.int32)
    boff = jnp.arange(B, dtype=jnp.int32)[:, None] * N
    idx_raster_g = (idx + boff).reshape(-1)
    idx5 = idx.reshape(B, 16, 8, 16, 8).transpose(0, 2, 4, 1, 3)
    idx_pool_g = (idx5.reshape(B, 64, 256) + boff[:, :, None]).reshape(-1)
    idx_local = idx.reshape(-1)

    g1 = ln1_g.reshape(1, C)
    b1 = ln1_b.reshape(1, C)
    g2 = ln2_g.reshape(1, C)
    b2 = ln2_b.reshape(1, C)
    bq2 = bq.reshape(1, C)
    bkv2 = bkv.reshape(1, 2 * C)
    bp2 = bp.reshape(1, C)
    bf12 = bf1.reshape(1, HID)
    bf22 = bf2.reshape(1, C)
    wskip2 = w_skip.reshape(1, HID)
    wdw9 = (w_dw[:, :, 0, :] * C1).reshape(9, HID)
    bdw2 = b_dw.reshape(1, HID)
    ones_t = jnp.ones((_K, CP), jnp.float32)
    zc_t = jnp.zeros((N // _NS, CP), jnp.float32)

    xn_f, q_f = _t1(xf, g1, b1, Wq, bq2)
    gkv = _sc_gather(xn_f, idx_pool_g, CP)
    x2_f = _t2(xf, q_f, gkv, Wkv, bkv2, Wp, bp2)
    hdn_f = _t3(x2_f, g2, b2, Wf1, bf12)
    ggrid = _sc_gather(hdn_f, idx_raster_g, HID).reshape(B, G, G, HID)
    m0, m1, m2 = _t4(ggrid, wdw9, bdw2)
    h0, h1, h2 = _sc_scatter(m0.reshape(M, CP), m1.reshape(M, CP),
                             m2.reshape(M, CP), idx_local, zc_t)
    cnt = _sc_count(idx_local, ones_t, zc_t)
    out_f = _t5(hdn_f, h0, h1, h2, cnt, x2_f, wskip2, Wf2, bf22)
    return out_f.reshape(B, N, C)


# speedup vs baseline: 6.2643x; 1.2282x over previous
import functools
import math

import jax
import jax.numpy as jnp
from jax import lax
from jax.experimental import pallas as pl
from jax.experimental.pallas import tpu as pltpu
from jax.experimental.pallas import tpu_sc as plsc

B, N, C = 4, 4096, 96
HEADS, HD = 2, 48
HID = 384
G = 128
NP = G * G
M = B * NP
SCALE = HD ** -0.5
C1 = 1.0 / (1.0 + 1e-6)

_NC, _NS = 2, 16
NW = _NC * _NS
_K = 64
CP = 128


_NB = 4


def _sc_gather(table, idxg, D):
    m_per_w = M // NW
    nchunk = m_per_w // _K

    def body(tab_hbm, idx_hbm, out_hbm, idx_v, rb0, rb1, rb2, rb3,
             g0, g1, g2, g3, w0, w1, w2, w3):
        rbs = (rb0, rb1, rb2, rb3)
        gsems = (g0, g1, g2, g3)
        wsems = (w0, w1, w2, w3)
        wid = lax.axis_index("s") * _NC + lax.axis_index("c")
        base = wid * m_per_w
        pltpu.sync_copy(idx_hbm.at[pl.ds(base, m_per_w)], idx_v)

        def gstart(i, b):
            pltpu.async_copy(tab_hbm.at[idx_v.at[pl.ds(i * _K, _K)]],
                             rbs[b], gsems[b])

        def wstart(i, b):
            pltpu.async_copy(rbs[b], out_hbm.at[pl.ds(base + i * _K, _K)],
                             wsems[b])

        for b in range(_NB):
            gstart(b, b)

        def step(j, carry):
            for b in range(_NB):
                i = j * _NB + b
                pltpu.make_async_copy(tab_hbm.at[pl.ds(0, _K)], rbs[b],
                                      gsems[b]).wait()
                wstart(i, b)
            for b in range(_NB):
                i_next = (j + 1) * _NB + b
                pltpu.make_async_copy(tab_hbm.at[pl.ds(0, _K)], rbs[b],
                                      wsems[b]).wait()

                @pl.when(i_next < nchunk)
                def _():
                    gstart(i_next, b)

            return carry

        lax.fori_loop(0, nchunk // _NB, step, 0)

    mesh = plsc.VectorSubcoreMesh(core_axis_name="c", subcore_axis_name="s")
    return pl.kernel(
        body,
        out_type=jax.ShapeDtypeStruct((M, D), jnp.float32),
        mesh=mesh,
        scratch_types=[pltpu.VMEM((m_per_w,), jnp.int32)]
        + [pltpu.VMEM((_K, D), jnp.float32)] * _NB
        + [pltpu.SemaphoreType.DMA] * (2 * _NB),
    )(table, idxg)


def _sc_scatter(s0, s1, s2, idx3, zeros_t):
    per_sub = NP // _NS
    nchunk = per_sub // _K
    rows_z = N // _NS

    def body(s0_hbm, s1_hbm, s2_hbm, idx_hbm, zeros_hbm,
             o0_hbm, o1_hbm, o2_hbm,
             idx_v, b00, b01, b10, b11, a0, a1,
             l0, l1, t0, t1):
        c = lax.axis_index("c")
        s = lax.axis_index("s")
        slabs = (s0_hbm, s1_hbm, s2_hbm)
        accs = (a0, a1)
        outs = (o0_hbm, o1_hbm, o2_hbm)
        bufs = ((b00, b01), (b10, b11))
        lsems = (l0, l1)
        ssems = (t0, t1)
        for bi in range(B // _NC):
            b = c + _NC * bi
            pltpu.sync_copy(idx_hbm.at[b * _NS + s], idx_v)
            base = b * NP + s * per_sub
            for group in ((0, 1), (2,)):
                ng = len(group)
                for gi in range(ng):
                    pltpu.sync_copy(zeros_hbm,
                                    accs[gi].at[pl.ds(s * rows_z, rows_z)])
                plsc.subcore_barrier()

                def lstart(i, t, group=group, ng=ng):
                    for gi in range(ng):
                        pltpu.async_copy(
                            slabs[group[gi]].at[pl.ds(base + i * _K, _K)],
                            bufs[t][gi], lsems[t])

                lstart(0, 0)
                lstart(1, 1)

                def step(j, carry, group=group, ng=ng, lstart=lstart):
                    for t in range(2):
                        i = j * 2 + t
                        for gi in range(ng):
                            pltpu.make_async_copy(
                                slabs[0].at[pl.ds(0, _K)], bufs[t][gi],
                                lsems[t]).wait()
                        for gi in range(ng):
                            pltpu.async_copy(bufs[t][gi],
                                             accs[gi].at[idx_v.at[i]],
                                             ssems[t], add=True)
                    for t in range(2):
                        i_next = (j + 1) * 2 + t
                        for gi in range(ng):
                            pltpu.make_async_copy(
                                slabs[0].at[pl.ds(0, _K)], bufs[t][gi],
                                ssems[t]).wait()

                        @pl.when(i_next < nchunk)
                        def _():
                            lstart(i_next, t)

                    return carry

                lax.fori_loop(0, nchunk // 2, step, 0)
                plsc.subcore_barrier()
                for gi in range(ng):
                    pltpu.sync_copy(
                        accs[gi].at[pl.ds(s * rows_z, rows_z)],
                        outs[group[gi]].at[pl.ds(b * N + s * rows_z, rows_z)])
                plsc.subcore_barrier()

    mesh = plsc.VectorSubcoreMesh(core_axis_name="c", subcore_axis_name="s")
    sds = jax.ShapeDtypeStruct((B * N, CP), jnp.float32)
    return pl.kernel(
        body,
        out_type=(sds, sds, sds),
        mesh=mesh,
        scratch_types=[pltpu.VMEM((nchunk, _K), jnp.int32)]
        + [pltpu.VMEM((_K, CP), jnp.float32)] * 4
        + [pltpu.VMEM_SHARED((N, CP), jnp.float32)] * 2
        + [pltpu.SemaphoreType.DMA] * 4,
    )(s0, s1, s2, idx3, zeros_t)


def _sc_count(idx3, ones_t, zeros_t):
    per_sub = NP // _NS
    nchunk = per_sub // _K
    rows_z = N // _NS

    def body(idx_hbm, ones_hbm, zeros_hbm, cnt_hbm, idx_v, ones_v, cnt_sh,
             asem):
        c = lax.axis_index("c")
        s = lax.axis_index("s")
        pltpu.sync_copy(ones_hbm, ones_v)
        for bi in range(B // _NC):
            b = c + _NC * bi
            pltpu.sync_copy(zeros_hbm, cnt_sh.at[pl.ds(s * rows_z, rows_z)])
            pltpu.sync_copy(idx_hbm.at[b * _NS + s], idx_v)
            plsc.subcore_barrier()

            def fire(i, carry):
                pltpu.async_copy(ones_v, cnt_sh.at[idx_v.at[i]], asem,
                                 add=True)
                return carry

            lax.fori_loop(0, nchunk, fire, 0)

            def drain(i, carry):
                pltpu.make_async_copy(ones_hbm, ones_v, asem).wait()
                return carry

            lax.fori_loop(0, nchunk, drain, 0)
            plsc.subcore_barrier()
            pltpu.sync_copy(cnt_sh.at[pl.ds(s * rows_z, rows_z)],
                            cnt_hbm.at[pl.ds(b * N + s * rows_z, rows_z)])
            plsc.subcore_barrier()

    mesh = plsc.VectorSubcoreMesh(core_axis_name="c", subcore_axis_name="s")
    return pl.kernel(
        body,
        out_type=jax.ShapeDtypeStruct((B * N, CP), jnp.float32),
        mesh=mesh,
        scratch_types=[
            pltpu.VMEM((nchunk, _K), jnp.int32),
            pltpu.VMEM((_K, CP), jnp.float32),
            pltpu.VMEM_SHARED((N, CP), jnp.float32),
            pltpu.SemaphoreType.DMA,
        ],
    )(idx3, ones_t, zeros_t)


def _ln(x_ref, g_ref, b_ref):
    xv = x_ref[...]
    m = jnp.mean(xv, axis=1, keepdims=True)
    d = xv - m
    v = jnp.mean(d * d, axis=1, keepdims=True)
    return d * lax.rsqrt(v + 1e-5) * g_ref[...] + b_ref[...]


def _t1_body(x_ref, g_ref, b_ref, w_ref, bias_ref, xn_ref, q_ref):
    xn = _ln(x_ref, g_ref, b_ref)
    xn_ref[...] = jnp.concatenate(
        [xn, jnp.zeros((xn.shape[0], CP - C), jnp.float32)], axis=1)
    q_ref[...] = (jnp.dot(xn, w_ref[...], preferred_element_type=jnp.float32)
                  + bias_ref[...])


def _t1(xf, g, bvec, Wq, bq):
    R = 512
    grid = (B * N) // R
    return pl.pallas_call(
        _t1_body,
        grid=(grid,),
        in_specs=[
            pl.BlockSpec((R, C), lambda i: (i, 0)),
            pl.BlockSpec((1, C), lambda i: (0, 0)),
            pl.BlockSpec((1, C), lambda i: (0, 0)),
            pl.BlockSpec((C, C), lambda i: (0, 0)),
            pl.BlockSpec((1, C), lambda i: (0, 0)),
        ],
        out_specs=[
            pl.BlockSpec((R, CP), lambda i: (i, 0)),
            pl.BlockSpec((R, C), lambda i: (i, 0)),
        ],
        out_shape=[
            jax.ShapeDtypeStruct((B * N, CP), jnp.float32),
            jax.ShapeDtypeStruct((B * N, C), jnp.float32),
        ],
    )(xf, g, bvec, Wq, bq)


def _t3_body(x_ref, g_ref, b_ref, w_ref, bias_ref, y_ref):
    xn = _ln(x_ref, g_ref, b_ref)
    y_ref[...] = (jnp.dot(xn, w_ref[...], preferred_element_type=jnp.float32)
                  + bias_ref[...])


def _t3(x2f, g, bvec, Wf1, bf1):
    R = 512
    grid = (B * N) // R
    return pl.pallas_call(
        _t3_body,
        grid=(grid,),
        in_specs=[
            pl.BlockSpec((R, C), lambda i: (i, 0)),
            pl.BlockSpec((1, C), lambda i: (0, 0)),
            pl.BlockSpec((1, C), lambda i: (0, 0)),
            pl.BlockSpec((C, HID), lambda i: (0, 0)),
            pl.BlockSpec((1, HID), lambda i: (0, 0)),
        ],
        out_specs=pl.BlockSpec((R, HID), lambda i: (i, 0)),
        out_shape=jax.ShapeDtypeStruct((B * N, HID), jnp.float32),
    )(x2f, g, bvec, Wf1, bf1)


def _t2_body(x_ref, q_ref, gkv_ref, wkv_ref, bkv_ref, wp_ref, bp_ref, x2_ref):
    def add(j, acc):
        return acc + gkv_ref[pl.ds(j * 256, 256), :C]

    ksum = lax.fori_loop(0, 64, add, jnp.zeros((256, C), jnp.float32))
    kv_tok = ksum * (C1 / 64.0)
    kv = (jnp.dot(kv_tok, wkv_ref[...], preferred_element_type=jnp.float32)
          + bkv_ref[...])
    q = q_ref[...]
    outs = []
    for h in range(HEADS):
        k_h = kv[:, h * HD:(h + 1) * HD]
        v_h = kv[:, C + h * HD:C + (h + 1) * HD]
        q_h = q[:, h * HD:(h + 1) * HD]
        logits = lax.dot_general(q_h, k_h, (((1,), (1,)), ((), ())),
                                 preferred_element_type=jnp.float32) * SCALE
        mx = jnp.max(logits, axis=1, keepdims=True)
        e = jnp.exp(logits - mx)
        p = e / jnp.sum(e, axis=1, keepdims=True)
        outs.append(jnp.dot(p, v_h, preferred_element_type=jnp.float32))
    o = jnp.concatenate(outs, axis=1)
    x2_ref[...] = (x_ref[...]
                   + jnp.dot(o, wp_ref[...], preferred_element_type=jnp.float32)
                   + bp_ref[...])


def _t2(xf, qf, gkv, Wkv, bkv, Wp, bp):
    return pl.pallas_call(
        _t2_body,
        grid=(B,),
        in_specs=[
            pl.BlockSpec((N, C), lambda b: (b, 0)),
            pl.BlockSpec((N, C), lambda b: (b, 0)),
            pl.BlockSpec((NP, CP), lambda b: (b, 0)),
            pl.BlockSpec((C, 2 * C), lambda b: (0, 0)),
            pl.BlockSpec((1, 2 * C), lambda b: (0, 0)),
            pl.BlockSpec((C, C), lambda b: (0, 0)),
            pl.BlockSpec((1, C), lambda b: (0, 0)),
        ],
        out_specs=pl.BlockSpec((N, C), lambda b: (b, 0)),
        out_shape=jax.ShapeDtypeStruct((B * N, C), jnp.float32),
    )(xf, qf, gkv, Wkv, bkv, Wp, bp)


def _t4_body(up_ref, mid_ref, dn_ref, w_ref, bd_ref, o0_ref, o1_ref, o2_ref):
    t = pl.program_id(1)
    nt = pl.num_programs(1)
    zrow = jnp.zeros((1, G, HID), jnp.float32)
    prev = jnp.where(t > 0, up_ref[0, 15:16], zrow)
    nxt = jnp.where(t < nt - 1, dn_ref[0, 0:1], zrow)
    padded = jnp.concatenate([prev, mid_ref[0], nxt], axis=0)
    acc = jnp.zeros((16, G, HID), jnp.float32)
    zcol = jnp.zeros((16, 1, HID), jnp.float32)
    for dr in range(3):
        rows = padded[dr:dr + 16]
        for dc in range(3):
            if dc == 0:
                sh = jnp.concatenate([zcol, rows[:, :-1]], axis=1)
            elif dc == 1:
                sh = rows
            else:
                sh = jnp.concatenate([rows[:, 1:], zcol], axis=1)
            acc = acc + sh * w_ref[dr * 3 + dc]
    acc = acc + bd_ref[0]
    o0_ref[0] = acc[:, :, 0:CP]
    o1_ref[0] = acc[:, :, CP:2 * CP]
    o2_ref[0] = acc[:, :, 2 * CP:3 * CP]


def _t4(ggrid, wdw9, bdw):
    RT = 16
    nt = G // RT
    spec = lambda f: pl.BlockSpec((1, RT, G, HID), f)
    ospec = pl.BlockSpec((1, RT, G, CP), lambda b, t: (b, t, 0, 0))
    osds = jax.ShapeDtypeStruct((B, G, G, CP), jnp.float32)
    return pl.pallas_call(
        _t4_body,
        grid=(B, nt),
        in_specs=[
            spec(lambda b, t: (b, jnp.maximum(t - 1, 0), 0, 0)),
            spec(lambda b, t: (b, t, 0, 0)),
            spec(lambda b, t: (b, jnp.minimum(t + 1, nt - 1), 0, 0)),
            pl.BlockSpec((9, HID), lambda b, t: (0, 0)),
            pl.BlockSpec((1, HID), lambda b, t: (0, 0)),
        ],
        out_specs=[ospec, ospec, ospec],
        out_shape=[osds, osds, osds],
    )(ggrid, ggrid, ggrid, wdw9, bdw)


def _t5_body(hdn_ref, h0_ref, h1_ref, h2_ref, cnt_ref, x2_ref, wskip_ref,
             wf2_ref, bf2_ref, out_ref):
    cntv = cnt_ref[:, 0:1]
    hsum = jnp.concatenate([h0_ref[...], h1_ref[...], h2_ref[...]], axis=1)
    htok = hsum / (cntv + 1e-6)
    a = hdn_ref[...] * wskip_ref[...] + htok
    gl = a * 0.5 * (1.0 + lax.erf(a * (2.0 ** -0.5)))
    out_ref[...] = (x2_ref[...]
                    + jnp.dot(gl, wf2_ref[...], preferred_element_type=jnp.float32)
                    + bf2_ref[...])


def _t5(hdn, h0, h1, h2, cnt, x2f, wskip, Wf2, bf2):
    R = 512
    grid = (B * N) // R
    return pl.pallas_call(
        _t5_body,
        grid=(grid,),
        in_specs=[
            pl.BlockSpec((R, HID), lambda i: (i, 0)),
            pl.BlockSpec((R, CP), lambda i: (i, 0)),
            pl.BlockSpec((R, CP), lambda i: (i, 0)),
            pl.BlockSpec((R, CP), lambda i: (i, 0)),
            pl.BlockSpec((R, CP), lambda i: (i, 0)),
            pl.BlockSpec((R, C), lambda i: (i, 0)),
            pl.BlockSpec((1, HID), lambda i: (0, 0)),
            pl.BlockSpec((HID, C), lambda i: (0, 0)),
            pl.BlockSpec((1, C), lambda i: (0, 0)),
        ],
        out_specs=pl.BlockSpec((R, C), lambda i: (i, 0)),
        out_shape=jax.ShapeDtypeStruct((B * N, C), jnp.float32),
    )(hdn, h0, h1, h2, cnt, x2f, wskip, Wf2, bf2)


def kernel(x, ln1_g, ln1_b, Wq, bq, Wkv, bkv, Wp, bp, ln2_g, ln2_b, Wf1, bf1,
           w_skip, w_dw, b_dw, Wf2, bf2, idx_token, H, W, H_init, W_init):
    xf = x.reshape(B * N, C)
    idx = idx_token.astype(jnp.int32)
    boff = jnp.arange(B, dtype=jnp.int32)[:, None] * N
    idx_raster_g = (idx + boff).reshape(-1)
    idx5 = idx.reshape(B, 16, 8, 16, 8).transpose(0, 2, 4, 1, 3)
    idx_pool_g = (idx5.reshape(B, 64, 256) + boff[:, :, None]).reshape(-1)
    idx_local = idx.reshape(-1)

    g1 = ln1_g.reshape(1, C)
    b1 = ln1_b.reshape(1, C)
    g2 = ln2_g.reshape(1, C)
    b2 = ln2_b.reshape(1, C)
    bq2 = bq.reshape(1, C)
    bkv2 = bkv.reshape(1, 2 * C)
    bp2 = bp.reshape(1, C)
    bf12 = bf1.reshape(1, HID)
    bf22 = bf2.reshape(1, C)
    wskip2 = w_skip.reshape(1, HID)
    wdw9 = (w_dw[:, :, 0, :] * C1).reshape(9, HID)
    bdw2 = b_dw.reshape(1, HID)
    ones_t = jnp.ones((_K, CP), jnp.float32)
    zc_t = jnp.zeros((N // _NS, CP), jnp.float32)

    xn_f, q_f = _t1(xf, g1, b1, Wq, bq2)
    gkv = _sc_gather(xn_f, idx_pool_g, CP)
    x2_f = _t2(xf, q_f, gkv, Wkv, bkv2, Wp, bp2)
    hdn_f = _t3(x2_f, g2, b2, Wf1, bf12)
    ggrid = _sc_gather(hdn_f, idx_raster_g, HID).reshape(B, G, G, HID)
    idx3 = idx_local.reshape(B * _NS, NP // (_NS * _K), _K)
    m0, m1, m2 = _t4(ggrid, wdw9, bdw2)
    h0, h1, h2 = _sc_scatter(m0.reshape(M, CP), m1.reshape(M, CP),
                             m2.reshape(M, CP), idx3, zc_t)
    cnt = _sc_count(idx3, ones_t, zc_t)
    out_f = _t5(hdn_f, h0, h1, h2, cnt, x2_f, wskip2, Wf2, bf22)
    return out_f.reshape(B, N, C)


# speedup vs baseline: 6.7728x; 1.0812x over previous
import jax
import jax.numpy as jnp
from jax import lax
from jax.experimental import pallas as pl
from jax.experimental.pallas import tpu as pltpu
from jax.experimental.pallas import tpu_sc as plsc

B, N, C = 4, 4096, 96
HEADS, HD = 2, 48
HID = 384
G = 128
NP = G * G
M = B * NP
SCALE = HD ** -0.5
C1 = 1.0 / (1.0 + 1e-6)

_NC, _NS = 2, 16
NW = _NC * _NS
_K = 64
CP = 128


_NB = 4


def _sc_gather(table, idxg, D, m_total):
    m_per_w = m_total // NW
    nchunk = m_per_w // _K

    def body(tab_hbm, idx_hbm, out_hbm, idx_v, rb0, rb1, rb2, rb3,
             g0, g1, g2, g3, w0, w1, w2, w3):
        rbs = (rb0, rb1, rb2, rb3)
        gsems = (g0, g1, g2, g3)
        wsems = (w0, w1, w2, w3)
        wid = lax.axis_index("s") * _NC + lax.axis_index("c")
        base = wid * m_per_w
        pltpu.sync_copy(idx_hbm.at[pl.ds(base, m_per_w)], idx_v)

        def gstart(i, b):
            pltpu.async_copy(tab_hbm.at[idx_v.at[pl.ds(i * _K, _K)]],
                             rbs[b], gsems[b])

        def wstart(i, b):
            pltpu.async_copy(rbs[b], out_hbm.at[pl.ds(base + i * _K, _K)],
                             wsems[b])

        for b in range(_NB):
            gstart(b, b)

        def step(j, carry):
            for b in range(_NB):
                i = j * _NB + b
                pltpu.make_async_copy(tab_hbm.at[pl.ds(0, _K)], rbs[b],
                                      gsems[b]).wait()
                wstart(i, b)
            for b in range(_NB):
                i_next = (j + 1) * _NB + b
                pltpu.make_async_copy(tab_hbm.at[pl.ds(0, _K)], rbs[b],
                                      wsems[b]).wait()

                @pl.when(i_next < nchunk)
                def _():
                    gstart(i_next, b)

            return carry

        lax.fori_loop(0, nchunk // _NB, step, 0)

    mesh = plsc.VectorSubcoreMesh(core_axis_name="c", subcore_axis_name="s")
    return pl.kernel(
        body,
        out_type=jax.ShapeDtypeStruct((m_total, D), jnp.float32),
        mesh=mesh,
        scratch_types=[pltpu.VMEM((m_per_w,), jnp.int32)]
        + [pltpu.VMEM((_K, D), jnp.float32)] * _NB
        + [pltpu.SemaphoreType.DMA] * (2 * _NB),
    )(table, idxg)


def _sc_scatter(s0, s1, s2, idx3, ones_t, zeros_t):
    per_sub = NP // _NS
    nchunk = per_sub // _K
    rows_z = N // _NS

    def body(s0_hbm, s1_hbm, s2_hbm, idx_hbm, ones_hbm, zeros_hbm,
             o0_hbm, o1_hbm, o2_hbm, cnt_hbm,
             idx_v, ones_v, b00, b01, b10, b11, a0, a1,
             l0, l1, t0, t1):
        c = lax.axis_index("c")
        s = lax.axis_index("s")
        slabs = (s0_hbm, s1_hbm, s2_hbm)
        accs = (a0, a1)
        outs = (o0_hbm, o1_hbm, o2_hbm, cnt_hbm)
        bufs = ((b00, b01), (b10, b11))
        lsems = (l0, l1)
        ssems = (t0, t1)
        pltpu.sync_copy(idx_hbm.at[c * _NS + s], idx_v)
        pltpu.sync_copy(ones_hbm, ones_v)
        base = c * NP + s * per_sub
        for group in ((0, 1), (2, 3)):
            ng = len(group)
            nl = sum(1 for k in group if k < 3)
            for gi in range(ng):
                pltpu.sync_copy(zeros_hbm,
                                accs[gi].at[pl.ds(s * rows_z, rows_z)])
            plsc.subcore_barrier()

            def lstart(i, t, group=group, nl=nl):
                for gi in range(nl):
                    pltpu.async_copy(
                        slabs[group[gi]].at[pl.ds(base + i * _K, _K)],
                        bufs[t][gi], lsems[t])

            lstart(0, 0)
            lstart(1, 1)

            def step(j, carry, group=group, ng=ng, nl=nl, lstart=lstart):
                for t in range(2):
                    i = j * 2 + t
                    for gi in range(nl):
                        pltpu.make_async_copy(
                            slabs[0].at[pl.ds(0, _K)], bufs[t][gi],
                            lsems[t]).wait()
                    for gi in range(ng):
                        src = bufs[t][gi] if group[gi] < 3 else ones_v
                        pltpu.async_copy(src, accs[gi].at[idx_v.at[i]],
                                         ssems[t], add=True)
                for t in range(2):
                    i_next = (j + 1) * 2 + t
                    for gi in range(ng):
                        dst = bufs[t][gi] if group[gi] < 3 else ones_v
                        pltpu.make_async_copy(
                            slabs[0].at[pl.ds(0, _K)], dst,
                            ssems[t]).wait()

                    @pl.when(i_next < nchunk)
                    def _():
                        lstart(i_next, t)

                return carry

            lax.fori_loop(0, nchunk // 2, step, 0)
            plsc.subcore_barrier()
            for gi in range(ng):
                pltpu.sync_copy(
                    accs[gi].at[pl.ds(s * rows_z, rows_z)],
                    outs[group[gi]].at[pl.ds(c * N + s * rows_z, rows_z)])
            plsc.subcore_barrier()

    mesh = plsc.VectorSubcoreMesh(core_axis_name="c", subcore_axis_name="s")
    sds = jax.ShapeDtypeStruct((2 * N, CP), jnp.float32)
    return pl.kernel(
        body,
        out_type=(sds, sds, sds, sds),
        mesh=mesh,
        scratch_types=[pltpu.VMEM((nchunk, _K), jnp.int32)]
        + [pltpu.VMEM((_K, CP), jnp.float32)] * 5
        + [pltpu.VMEM_SHARED((N, CP), jnp.float32)] * 2
        + [pltpu.SemaphoreType.DMA] * 4,
    )(s0, s1, s2, idx3, ones_t, zeros_t)


def _ln(x_ref, g_ref, b_ref):
    xv = x_ref[...]
    m = jnp.mean(xv, axis=1, keepdims=True)
    d = xv - m
    v = jnp.mean(d * d, axis=1, keepdims=True)
    return d * lax.rsqrt(v + 1e-5) * g_ref[...] + b_ref[...]


def _t1_body(x_ref, g_ref, b_ref, w_ref, bias_ref, xn_ref, q_ref):
    xn = _ln(x_ref, g_ref, b_ref)
    xn_ref[...] = jnp.concatenate(
        [xn, jnp.zeros((xn.shape[0], CP - C), jnp.float32)], axis=1)
    q_ref[...] = (jnp.dot(xn, w_ref[...], preferred_element_type=jnp.float32)
                  + bias_ref[...])


def _t1(xf, g, bvec, Wq, bq):
    R = 512
    grid = (B * N) // R
    return pl.pallas_call(
        _t1_body,
        grid=(grid,),
        in_specs=[
            pl.BlockSpec((R, C), lambda i: (i, 0)),
            pl.BlockSpec((1, C), lambda i: (0, 0)),
            pl.BlockSpec((1, C), lambda i: (0, 0)),
            pl.BlockSpec((C, C), lambda i: (0, 0)),
            pl.BlockSpec((1, C), lambda i: (0, 0)),
        ],
        out_specs=[
            pl.BlockSpec((R, CP), lambda i: (i, 0)),
            pl.BlockSpec((R, C), lambda i: (i, 0)),
        ],
        out_shape=[
            jax.ShapeDtypeStruct((B * N, CP), jnp.float32),
            jax.ShapeDtypeStruct((B * N, C), jnp.float32),
        ],
    )(xf, g, bvec, Wq, bq)


def _t3_body(x_ref, g_ref, b_ref, w_ref, bias_ref, y_ref):
    xn = _ln(x_ref, g_ref, b_ref)
    y_ref[...] = (jnp.dot(xn, w_ref[...], preferred_element_type=jnp.float32)
                  + bias_ref[...])


def _t3(x2f, g, bvec, Wf1, bf1):
    R = 512
    grid = x2f.shape[0] // R
    return pl.pallas_call(
        _t3_body,
        grid=(grid,),
        in_specs=[
            pl.BlockSpec((R, C), lambda i: (i, 0)),
            pl.BlockSpec((1, C), lambda i: (0, 0)),
            pl.BlockSpec((1, C), lambda i: (0, 0)),
            pl.BlockSpec((C, HID), lambda i: (0, 0)),
            pl.BlockSpec((1, HID), lambda i: (0, 0)),
        ],
        out_specs=pl.BlockSpec((R, HID), lambda i: (i, 0)),
        out_shape=jax.ShapeDtypeStruct((x2f.shape[0], HID), jnp.float32),
    )(x2f, g, bvec, Wf1, bf1)


def _t2_body(x_ref, q_ref, gkv_ref, wkv_ref, bkv_ref, wp_ref, bp_ref, x2_ref):
    def add(j, acc):
        return acc + gkv_ref[pl.ds(j * 256, 256), :C]

    ksum = lax.fori_loop(0, 64, add, jnp.zeros((256, C), jnp.float32))
    kv_tok = ksum * (C1 / 64.0)
    kv = (jnp.dot(kv_tok, wkv_ref[...], preferred_element_type=jnp.float32)
          + bkv_ref[...])
    q = q_ref[...]
    outs = []
    for h in range(HEADS):
        k_h = kv[:, h * HD:(h + 1) * HD]
        v_h = kv[:, C + h * HD:C + (h + 1) * HD]
        q_h = q[:, h * HD:(h + 1) * HD]
        logits = lax.dot_general(q_h, k_h, (((1,), (1,)), ((), ())),
                                 preferred_element_type=jnp.float32) * SCALE
        mx = jnp.max(logits, axis=1, keepdims=True)
        e = jnp.exp(logits - mx)
        p = e / jnp.sum(e, axis=1, keepdims=True)
        outs.append(jnp.dot(p, v_h, preferred_element_type=jnp.float32))
    o = jnp.concatenate(outs, axis=1)
    x2_ref[...] = (x_ref[...]
                   + jnp.dot(o, wp_ref[...], preferred_element_type=jnp.float32)
                   + bp_ref[...])


def _t2(xf, qf, gkv, Wkv, bkv, Wp, bp):
    return pl.pallas_call(
        _t2_body,
        grid=(B,),
        in_specs=[
            pl.BlockSpec((N, C), lambda b: (b, 0)),
            pl.BlockSpec((N, C), lambda b: (b, 0)),
            pl.BlockSpec((NP, CP), lambda b: (b, 0)),
            pl.BlockSpec((C, 2 * C), lambda b: (0, 0)),
            pl.BlockSpec((1, 2 * C), lambda b: (0, 0)),
            pl.BlockSpec((C, C), lambda b: (0, 0)),
            pl.BlockSpec((1, C), lambda b: (0, 0)),
        ],
        out_specs=pl.BlockSpec((N, C), lambda b: (b, 0)),
        out_shape=jax.ShapeDtypeStruct((B * N, C), jnp.float32),
    )(xf, qf, gkv, Wkv, bkv, Wp, bp)


def _t4_body(mid_ref, up_ref, dn_ref, w_ref, bd_ref, o0_ref, o1_ref, o2_ref):
    padded = jnp.concatenate([up_ref[0], mid_ref[0], dn_ref[0]],
                             axis=0)
    acc = jnp.zeros((16, G, HID), jnp.float32)
    zcol = jnp.zeros((16, 1, HID), jnp.float32)
    for dr in range(3):
        rows = padded[dr:dr + 16]
        for dc in range(3):
            if dc == 0:
                sh = jnp.concatenate([zcol, rows[:, :-1]], axis=1)
            elif dc == 1:
                sh = rows
            else:
                sh = jnp.concatenate([rows[:, 1:], zcol], axis=1)
            acc = acc + sh * w_ref[dr * 3 + dc]
    acc = acc + bd_ref[0]
    o0_ref[0] = acc[:, :, 0:CP]
    o1_ref[0] = acc[:, :, CP:2 * CP]
    o2_ref[0] = acc[:, :, 2 * CP:3 * CP]


def _t4(ggrid, up_e, dn_e, wdw9, bdw):
    RT = 16
    nt = G // RT
    Bb = ggrid.shape[0]
    ospec = pl.BlockSpec((1, RT, G, CP), lambda b, t: (b, t, 0, 0))
    osds = jax.ShapeDtypeStruct((Bb, G, G, CP), jnp.float32)
    return pl.pallas_call(
        _t4_body,
        grid=(Bb, nt),
        in_specs=[
            pl.BlockSpec((1, RT, G, HID), lambda b, t: (b, t, 0, 0)),
            pl.BlockSpec((1, 1, G, HID), lambda b, t: (b, t, 0, 0)),
            pl.BlockSpec((1, 1, G, HID), lambda b, t: (b, t, 0, 0)),
            pl.BlockSpec((9, HID), lambda b, t: (0, 0)),
            pl.BlockSpec((1, HID), lambda b, t: (0, 0)),
        ],
        out_specs=[ospec, ospec, ospec],
        out_shape=[osds, osds, osds],
    )(ggrid, up_e, dn_e, wdw9, bdw)


def _t5_body(hdn_ref, h0_ref, h1_ref, h2_ref, cnt_ref, x2_ref, wskip_ref,
             wf2_ref, bf2_ref, out_ref):
    cntv = cnt_ref[:, 0:1]
    hsum = jnp.concatenate([h0_ref[...], h1_ref[...], h2_ref[...]], axis=1)
    htok = hsum / (cntv + 1e-6)
    a = hdn_ref[...] * wskip_ref[...] + htok
    gl = a * 0.5 * (1.0 + lax.erf(a * (2.0 ** -0.5)))
    out_ref[...] = (x2_ref[...]
                    + jnp.dot(gl, wf2_ref[...], preferred_element_type=jnp.float32)
                    + bf2_ref[...])


def _t5(hdn, h0, h1, h2, cnt, x2f, wskip, Wf2, bf2):
    R = 512
    grid = x2f.shape[0] // R
    return pl.pallas_call(
        _t5_body,
        grid=(grid,),
        in_specs=[
            pl.BlockSpec((R, HID), lambda i: (i, 0)),
            pl.BlockSpec((R, CP), lambda i: (i, 0)),
            pl.BlockSpec((R, CP), lambda i: (i, 0)),
            pl.BlockSpec((R, CP), lambda i: (i, 0)),
            pl.BlockSpec((R, CP), lambda i: (i, 0)),
            pl.BlockSpec((R, C), lambda i: (i, 0)),
            pl.BlockSpec((1, HID), lambda i: (0, 0)),
            pl.BlockSpec((HID, C), lambda i: (0, 0)),
            pl.BlockSpec((1, C), lambda i: (0, 0)),
        ],
        out_specs=pl.BlockSpec((R, C), lambda i: (i, 0)),
        out_shape=jax.ShapeDtypeStruct((x2f.shape[0], C), jnp.float32),
    )(hdn, h0, h1, h2, cnt, x2f, wskip, Wf2, bf2)


def kernel(x, ln1_g, ln1_b, Wq, bq, Wkv, bkv, Wp, bp, ln2_g, ln2_b, Wf1, bf1,
           w_skip, w_dw, b_dw, Wf2, bf2, idx_token, H, W, H_init, W_init):
    xf = x.reshape(B * N, C)
    idx = idx_token.astype(jnp.int32)
    boff = jnp.arange(B, dtype=jnp.int32)[:, None] * N
    idx_raster_g = (idx + boff).reshape(-1)
    idx5 = idx.reshape(B, 16, 8, 16, 8).transpose(0, 2, 4, 1, 3)
    idx_pool_g = (idx5.reshape(B, 64, 256) + boff[:, :, None]).reshape(-1)
    idx_local = idx.reshape(-1)

    g1 = ln1_g.reshape(1, C)
    b1 = ln1_b.reshape(1, C)
    g2 = ln2_g.reshape(1, C)
    b2 = ln2_b.reshape(1, C)
    bq2 = bq.reshape(1, C)
    bkv2 = bkv.reshape(1, 2 * C)
    bp2 = bp.reshape(1, C)
    bf12 = bf1.reshape(1, HID)
    bf22 = bf2.reshape(1, C)
    wskip2 = w_skip.reshape(1, HID)
    wdw9 = (w_dw[:, :, 0, :] * C1).reshape(9, HID)
    bdw2 = b_dw.reshape(1, HID)
    ones_t = jnp.ones((_K, CP), jnp.float32)
    zc_t = jnp.zeros((N // _NS, CP), jnp.float32)

    xn_f, q_f = _t1(xf, g1, b1, Wq, bq2)
    gkv = _sc_gather(xn_f, idx_pool_g, CP, M)
    x2_f = _t2(xf, q_f, gkv, Wkv, bkv2, Wp, bp2)

    outs = []
    pair_off = jnp.arange(2, dtype=jnp.int32)[:, None] * N
    for h in range(2):
        sl = slice(h * 2 * N, (h + 1) * 2 * N)
        x2_h = x2_f[sl]
        hdn_h = _t3(x2_h, g2, b2, Wf1, bf12)
        idxg_h = (idx[2 * h:2 * h + 2] + pair_off).reshape(-1)
        gg = _sc_gather(hdn_h, idxg_h, HID, 2 * NP).reshape(2, G, G, HID)
        zrow = jnp.zeros((2, 1, G, HID), jnp.float32)
        up_e = jnp.concatenate([zrow, gg[:, 15:G - 1:16]], axis=1)
        dn_e = jnp.concatenate([gg[:, 16:G:16], zrow], axis=1)
        m0, m1, m2 = _t4(gg, up_e, dn_e, wdw9, bdw2)
        idx3h = idx[2 * h:2 * h + 2].reshape(2 * _NS, NP // (_NS * _K), _K)
        h0, h1, h2, cnt_h = _sc_scatter(m0.reshape(2 * NP, CP),
                                        m1.reshape(2 * NP, CP),
                                        m2.reshape(2 * NP, CP), idx3h,
                                        ones_t, zc_t)
        outs.append(_t5(hdn_h, h0, h1, h2, cnt_h, x2_h, wskip2, Wf2, bf22))
    return jnp.concatenate(outs, axis=0).reshape(B, N, C)
